# baseline (speedup 1.0000x reference)
import functools

import jax
import jax.numpy as jnp
from jax.experimental import pallas as pl

BATCH = 4096
HIST_LEN = 200
EXPAND_DIM = 64


def _body(ts_ref, emb_ref, out_ref):
    ts = ts_ref[...]
    b, h = ts.shape
    tsi = ts[:, :, None]
    tsj = ts[:, None, :]
    ii = jax.lax.broadcasted_iota(jnp.int32, (b, h, h), 1)
    jj = jax.lax.broadcasted_iota(jnp.int32, (b, h, h), 2)
    less = (tsj < tsi) | ((tsj == tsi) & (jj < ii))
    rank = jnp.sum(less.astype(jnp.float32), axis=2)
    kk = jax.lax.broadcasted_iota(jnp.int32, (b, h, h), 1).astype(jnp.float32)
    onehot = (rank[:, None, :] == kk).astype(jnp.float32)
    emb = emb_ref[...]
    out_ref[...] = jax.lax.dot_general(
        onehot, emb, (((2,), (0,)), ((), ())),
        preferred_element_type=jnp.float32,
    )


@jax.jit
def kernel(ts, pos_embeddings):
    B = 16
    grid = (BATCH // B,)
    return pl.pallas_call(
        _body,
        grid=grid,
        in_specs=[
            pl.BlockSpec((B, HIST_LEN), lambda i: (i, 0)),
            pl.BlockSpec((HIST_LEN, EXPAND_DIM), lambda i: (0, 0)),
        ],
        out_specs=pl.BlockSpec((B, HIST_LEN, EXPAND_DIM), lambda i: (i, 0, 0)),
        out_shape=jax.ShapeDtypeStruct((BATCH, HIST_LEN, EXPAND_DIM), jnp.float32),
    )(ts, pos_embeddings)


# speedup vs baseline: 21.1634x; 21.1634x over previous
import jax
import jax.numpy as jnp
from jax.experimental import pallas as pl

BATCH = 4096
HIST_LEN = 200
EXPAND_DIM = 64


def _body(ts_ref, tst_ref, emb_ref, out_ref):
    ts = ts_ref[...]
    tsj = tst_ref[...]
    b, h = ts.shape
    tsi = ts[:, None, :]
    jj = jax.lax.broadcasted_iota(jnp.int32, (b, h, h), 1)
    ii = jax.lax.broadcasted_iota(jnp.int32, (b, h, h), 2)
    less = (tsj < tsi) | ((tsj == tsi) & (jj < ii))
    rank = jnp.sum(less.astype(jnp.float32), axis=1)
    kk = jax.lax.broadcasted_iota(jnp.int32, (b, h, h), 1).astype(jnp.float32)
    onehot = (rank[:, None, :] == kk).astype(jnp.float32)
    out_ref[...] = jax.lax.dot_general(
        onehot, emb_ref[...], (((2,), (0,)), ((), ())),
        preferred_element_type=jnp.float32,
    )


@jax.jit
def kernel(ts, pos_embeddings):
    B = 16
    grid = (BATCH // B,)
    ts_t = ts[..., None]
    return pl.pallas_call(
        _body,
        grid=grid,
        in_specs=[
            pl.BlockSpec((B, HIST_LEN), lambda i: (i, 0)),
            pl.BlockSpec((B, HIST_LEN, 1), lambda i: (i, 0, 0)),
            pl.BlockSpec((HIST_LEN, EXPAND_DIM), lambda i: (0, 0)),
        ],
        out_specs=pl.BlockSpec((B, HIST_LEN, EXPAND_DIM), lambda i: (i, 0, 0)),
        out_shape=jax.ShapeDtypeStruct((BATCH, HIST_LEN, EXPAND_DIM), jnp.float32),
    )(ts, ts_t, pos_embeddings)


# speedup vs baseline: 23.2328x; 1.0978x over previous
import jax
import jax.numpy as jnp
from jax.experimental import pallas as pl

BATCH = 4096
HIST_LEN = 200
EXPAND_DIM = 64


def _body(ts_ref, tst_ref, emb_ref, out_ref):
    ts = ts_ref[...]
    tsj = tst_ref[...]
    b, h = ts.shape
    tsi = ts[:, None, :]
    jj = jax.lax.broadcasted_iota(jnp.int32, (b, h, h), 1)
    ii = jax.lax.broadcasted_iota(jnp.int32, (b, h, h), 2)
    less = (tsj < tsi) | ((tsj == tsi) & (jj < ii))
    rank = jnp.sum(less.astype(jnp.float32), axis=1)
    kk = jax.lax.broadcasted_iota(jnp.int32, (b, h, h), 1).astype(jnp.float32)
    onehot = (rank[:, None, :] == kk).astype(jnp.float32)
    out_ref[...] = jax.lax.dot_general(
        onehot, emb_ref[...], (((2,), (0,)), ((), ())),
        preferred_element_type=jnp.float32,
    )


@jax.jit
def kernel(ts, pos_embeddings):
    B = 32
    grid = (BATCH // B,)
    ts_t = ts[..., None]
    return pl.pallas_call(
        _body,
        grid=grid,
        in_specs=[
            pl.BlockSpec((B, HIST_LEN), lambda i: (i, 0)),
            pl.BlockSpec((B, HIST_LEN, 1), lambda i: (i, 0, 0)),
            pl.BlockSpec((HIST_LEN, EXPAND_DIM), lambda i: (0, 0)),
        ],
        out_specs=pl.BlockSpec((B, HIST_LEN, EXPAND_DIM), lambda i: (i, 0, 0)),
        out_shape=jax.ShapeDtypeStruct((BATCH, HIST_LEN, EXPAND_DIM), jnp.float32),
    )(ts, ts_t, pos_embeddings)


# speedup vs baseline: 26.3673x; 1.1349x over previous
import jax
import jax.numpy as jnp
from jax.experimental import pallas as pl

BATCH = 4096
HIST_LEN = 200
EXPAND_DIM = 64


def _body(ts_ref, tst_ref, emb_ref, out_ref):
    ts = ts_ref[...]
    tsj = tst_ref[...]
    b, h = ts.shape
    tsi = ts[:, None, :]
    jj = jax.lax.broadcasted_iota(jnp.int32, (b, h, h), 1)
    ii = jax.lax.broadcasted_iota(jnp.int32, (b, h, h), 2)
    less = (tsj < tsi) | ((tsj == tsi) & (jj < ii))
    rank = jnp.sum(less.astype(jnp.float32), axis=1)
    kk2 = jax.lax.broadcasted_iota(jnp.int32, (b, h // 2, h), 1)
    rank_b = rank[:, None, :]
    oh_even = (rank_b == (2 * kk2).astype(jnp.float32)).astype(jnp.float32)
    oh_odd = (rank_b == (2 * kk2 + 1).astype(jnp.float32)).astype(jnp.float32)
    emb = emb_ref[...]
    dn = (((2,), (0,)), ((), ()))
    r_even = jax.lax.dot_general(oh_even, emb, dn,
                                 preferred_element_type=jnp.float32)
    r_odd = jax.lax.dot_general(oh_odd, emb, dn,
                                preferred_element_type=jnp.float32)
    out_ref[...] = jnp.concatenate([r_even, r_odd], axis=2)


@jax.jit
def kernel(ts, pos_embeddings):
    B = 32
    grid = (BATCH // B,)
    ts_t = ts[..., None]
    return pl.pallas_call(
        _body,
        grid=grid,
        in_specs=[
            pl.BlockSpec((B, HIST_LEN), lambda i: (i, 0)),
            pl.BlockSpec((B, HIST_LEN, 1), lambda i: (i, 0, 0)),
            pl.BlockSpec((HIST_LEN, EXPAND_DIM), lambda i: (0, 0)),
        ],
        out_specs=pl.BlockSpec((B, HIST_LEN // 2, 2 * EXPAND_DIM),
                               lambda i: (i, 0, 0)),
        out_shape=jax.ShapeDtypeStruct(
            (BATCH, HIST_LEN // 2, 2 * EXPAND_DIM), jnp.float32),
    )(ts, ts_t, pos_embeddings).reshape(BATCH, HIST_LEN, EXPAND_DIM)
